# Initial kernel scaffold; baseline (speedup 1.0000x reference)
import jax
import jax.numpy as jnp
from jax.experimental import pallas as pl


def kernel(x, edge_index, W1, b1, W2, b2):
    raise NotImplementedError("write your pallas kernel here")


# speedup vs baseline: 1.0078x; 1.0078x over previous
import jax
import jax.numpy as jnp
from jax.experimental import pallas as pl


def _ew_body(a_ref, b_ref, o_ref):
    o_ref[...] = a_ref[...] + b_ref[...]


def _gcn_conv(x, edge_index, W, b):
    n = x.shape[0]
    loop = jnp.arange(n, dtype=edge_index.dtype)
    src = jnp.concatenate([edge_index[0], loop])
    dst = jnp.concatenate([edge_index[1], loop])
    deg = jnp.zeros((n,), dtype=x.dtype).at[dst].add(1.0)
    deg_inv_sqrt = jnp.where(deg > 0, 1.0 / jnp.sqrt(deg), 0.0)
    norm = deg_inv_sqrt[src] * deg_inv_sqrt[dst]
    h = x @ W
    msg = h[src] * norm[:, None]
    out = jnp.zeros((n, W.shape[1]), dtype=x.dtype).at[dst].add(msg)
    return out, b


def kernel(x, edge_index, W1, b1, W2, b2):
    out1, bb1 = _gcn_conv(x, edge_index, W1, b1)
    h = jax.nn.relu(
        pl.pallas_call(
            _ew_body,
            out_shape=jax.ShapeDtypeStruct(out1.shape, out1.dtype),
        )(out1, jnp.broadcast_to(bb1, out1.shape))
    )
    out2, bb2 = _gcn_conv(h, edge_index, W2, b2)
    return pl.pallas_call(
        _ew_body,
        out_shape=jax.ShapeDtypeStruct(out2.shape, out2.dtype),
    )(out2, jnp.broadcast_to(bb2, out2.shape))


# speedup vs baseline: 10.9335x; 10.8488x over previous
import functools

import jax
import jax.numpy as jnp
from jax import lax
from jax.experimental import pallas as pl
from jax.experimental.pallas import tpu as pltpu
from jax.experimental.pallas import tpu_sc as plsc

_N = 10000
_E = 320000
_NP = 10240
_CSZ = 128
_NCH = 80
_NC = 2
_NS = 16
_NW = _NC * _NS
_RPT = _NP // _NS

_mesh = plsc.VectorSubcoreMesh(core_axis_name="c", subcore_axis_name="s")


def _zero_fill(buf, nrow, ncol):
    def row(r, _):
        for j in range(ncol // 16):
            buf[r, pl.ds(j * 16, 16)] = jnp.zeros((16,), jnp.float32)
        return 0
    lax.fori_loop(0, nrow, row, 0)


@functools.partial(
    pl.kernel,
    out_type=jax.ShapeDtypeStruct((_NC, _NP, 16), jnp.float32),
    mesh=_mesh,
    scratch_types=[
        pltpu.VMEM((_NCH, _CSZ), jnp.int32),
        pltpu.VMEM((_CSZ, 16), jnp.float32),
        pltpu.VMEM_SHARED((_NP, 16), jnp.float32),
    ],
)
def _sc_deg(dst3, degp, idx_all, buf, acc):
    c = lax.axis_index("c")
    s = lax.axis_index("s")
    wid = s * _NC + c
    _zero_fill(buf, _CSZ, 16)
    for k in range(_RPT // _CSZ):
        pltpu.sync_copy(buf, acc.at[pl.ds(s * _RPT + k * _CSZ, _CSZ)])
    def row(r, _):
        buf[r, :] = jnp.ones((16,), jnp.float32)
        return 0
    lax.fori_loop(0, _CSZ, row, 0)
    pltpu.sync_copy(dst3.at[wid], idx_all)
    plsc.subcore_barrier()
    def chunk(ch, _):
        pltpu.sync_copy(buf, acc.at[idx_all.at[ch]], add=True)
        return 0
    lax.fori_loop(0, _NCH, chunk, 0)
    plsc.subcore_barrier()
    pltpu.sync_copy(acc.at[pl.ds(s * _RPT, _RPT)],
                    degp.at[c, pl.ds(s * _RPT, _RPT)])


def _make_spmm(d):
    @functools.partial(
        pl.kernel,
        out_type=jax.ShapeDtypeStruct((_NC, _NP, d), jnp.float32),
        mesh=_mesh,
        scratch_types=[
            pltpu.VMEM((_NCH, _CSZ), jnp.int32),
            pltpu.VMEM((_NCH, _CSZ), jnp.int32),
            pltpu.VMEM((_CSZ, d), jnp.float32),
            pltpu.VMEM_SHARED((_NP, d), jnp.float32),
            pltpu.SemaphoreType.DMA,
        ],
        compiler_params=pltpu.CompilerParams(use_tc_tiling_on_sc=False),
    )
    def _sc_spmm(hp, src3, dst3, outp, src_all, dst_all, rows, acc, sem):
        c = lax.axis_index("c")
        s = lax.axis_index("s")
        wid = s * _NC + c
        _zero_fill(rows, _CSZ, d)
        for k in range(_RPT // _CSZ):
            pltpu.sync_copy(rows, acc.at[pl.ds(s * _RPT + k * _CSZ, _CSZ)])
        pltpu.sync_copy(src3.at[wid], src_all)
        pltpu.sync_copy(dst3.at[wid], dst_all)
        plsc.subcore_barrier()
        def chunk(ch, _):
            pltpu.async_copy(hp.at[src_all.at[ch]], rows, sem).wait()
            pltpu.sync_copy(rows, acc.at[dst_all.at[ch]], add=True)
            return 0
        lax.fori_loop(0, _NCH, chunk, 0)
        plsc.subcore_barrier()
        pltpu.sync_copy(acc.at[pl.ds(s * _RPT, _RPT)],
                        outp.at[c, pl.ds(s * _RPT, _RPT)])
    return _sc_spmm


_sc_spmm_128 = _make_spmm(128)
_sc_spmm_64 = _make_spmm(64)


_R = 1000


def _tc1_body(x_ref, w_ref, d0_ref, d1_ref, hp_ref):
    deg = d0_ref[0, :, 0:1] + d1_ref[0, :, 0:1] + 1.0
    dis = lax.rsqrt(deg)
    h = jnp.dot(x_ref[...], w_ref[...], preferred_element_type=jnp.float32)
    hp_ref[...] = h * dis


def _tc_mid_body(s0_ref, s1_ref, hp_ref, d0_ref, d1_ref, b_ref, w_ref, out_ref):
    deg = d0_ref[0, :, 0:1] + d1_ref[0, :, 0:1] + 1.0
    dis = lax.rsqrt(deg)
    tot = s0_ref[0] + s1_ref[0] + hp_ref[...]
    h = jnp.maximum(tot * dis + b_ref[...], 0.0)
    out_ref[...] = jnp.dot(h, w_ref[...], preferred_element_type=jnp.float32) * dis


def _tc_out_body(s0_ref, s1_ref, hp_ref, d0_ref, d1_ref, b_ref, out_ref):
    deg = d0_ref[0, :, 0:1] + d1_ref[0, :, 0:1] + 1.0
    dis = lax.rsqrt(deg)
    out_ref[...] = (s0_ref[0] + s1_ref[0] + hp_ref[...]) * dis + b_ref[...]


def _row_spec(d):
    return pl.BlockSpec((_R, d), lambda i: (i, 0))


def _full_spec(shape):
    return pl.BlockSpec(shape, lambda i: tuple(0 for _ in shape))


def kernel(x, edge_index, W1, b1, W2, b2):
    src = edge_index[0].astype(jnp.int32)
    dst = edge_index[1].astype(jnp.int32)
    pad = _NW * _NCH * _CSZ - _E
    src3 = jnp.concatenate([src, jnp.zeros((pad,), jnp.int32)]).reshape(_NW, _NCH, _CSZ)
    dst3 = jnp.concatenate([dst, jnp.full((pad,), _N, jnp.int32)]).reshape(_NW, _NCH, _CSZ)

    degp = _sc_deg(dst3)

    def _part_spec(d, j):
        return pl.BlockSpec((1, _R, d), lambda i, j=j: (j, i, 0))

    hp1 = pl.pallas_call(
        _tc1_body,
        grid=(_N // _R,),
        in_specs=[_row_spec(128), _full_spec((128, 128)),
                  _part_spec(16, 0), _part_spec(16, 1)],
        out_specs=_row_spec(128),
        out_shape=jax.ShapeDtypeStruct((_N, 128), jnp.float32),
    )(x, W1, degp, degp)

    s1 = _sc_spmm_128(hp1, src3, dst3)

    hp2 = pl.pallas_call(
        _tc_mid_body,
        grid=(_N // _R,),
        in_specs=[
            _part_spec(128, 0),
            _part_spec(128, 1),
            _row_spec(128),
            _part_spec(16, 0),
            _part_spec(16, 1),
            _full_spec((1, 128)),
            _full_spec((128, 64)),
        ],
        out_specs=_row_spec(64),
        out_shape=jax.ShapeDtypeStruct((_N, 64), jnp.float32),
    )(s1, s1, hp1, degp, degp, b1.reshape(1, 128), W2)

    s2 = _sc_spmm_64(hp2, src3, dst3)

    out = pl.pallas_call(
        _tc_out_body,
        grid=(_N // _R,),
        in_specs=[
            _part_spec(64, 0),
            _part_spec(64, 1),
            _row_spec(64),
            _part_spec(16, 0),
            _part_spec(16, 1),
            _full_spec((1, 64)),
        ],
        out_specs=_row_spec(64),
        out_shape=jax.ShapeDtypeStruct((_N, 64), jnp.float32),
    )(s2, s2, hp2, degp, degp, b2.reshape(1, 64))
    return out


# speedup vs baseline: 14.3127x; 1.3091x over previous
import functools

import jax
import jax.numpy as jnp
from jax import lax
from jax.experimental import pallas as pl
from jax.experimental.pallas import tpu as pltpu
from jax.experimental.pallas import tpu_sc as plsc

_N = 10000
_E = 320000
_NP = 10240
_CSZ = 96
_NCH = 106
_NC = 2
_NS = 16
_NW = _NC * _NS
_RPT = _NP // _NS

_mesh = plsc.VectorSubcoreMesh(core_axis_name="c", subcore_axis_name="s")


def _zero_fill(buf, nrow, ncol):
    def row(r, _):
        for j in range(ncol // 16):
            buf[r, pl.ds(j * 16, 16)] = jnp.zeros((16,), jnp.float32)
        return 0
    lax.fori_loop(0, nrow, row, 0)


def _zero_acc_slice(buf, acc, base):
    nfull = _RPT // _CSZ
    rem = _RPT - nfull * _CSZ
    for k in range(nfull):
        pltpu.sync_copy(buf, acc.at[pl.ds(base + k * _CSZ, _CSZ)])
    if rem:
        pltpu.sync_copy(buf.at[pl.ds(0, rem)],
                        acc.at[pl.ds(base + nfull * _CSZ, rem)])


@functools.partial(
    pl.kernel,
    out_type=jax.ShapeDtypeStruct((_NC, _NP, 16), jnp.float32),
    mesh=_mesh,
    scratch_types=[
        pltpu.VMEM((_NCH, _CSZ), jnp.int32),
        pltpu.VMEM((_CSZ, 16), jnp.float32),
        pltpu.VMEM_SHARED((_NP, 16), jnp.float32),
    ],
)
def _sc_deg(dst3, degp, idx_all, buf, acc):
    c = lax.axis_index("c")
    s = lax.axis_index("s")
    wid = s * _NC + c
    _zero_fill(buf, _CSZ, 16)
    _zero_acc_slice(buf, acc, s * _RPT)
    def row(r, _):
        buf[r, :] = jnp.ones((16,), jnp.float32)
        return 0
    lax.fori_loop(0, _CSZ, row, 0)
    pltpu.sync_copy(dst3.at[wid], idx_all)
    plsc.subcore_barrier()
    def chunk(ch, _):
        pltpu.sync_copy(buf, acc.at[idx_all.at[ch]], add=True)
        return 0
    lax.fori_loop(0, _NCH, chunk, 0)
    plsc.subcore_barrier()
    pltpu.sync_copy(acc.at[pl.ds(s * _RPT, _RPT)],
                    degp.at[c, pl.ds(s * _RPT, _RPT)])


def _make_spmm(d):
    @functools.partial(
        pl.kernel,
        out_type=jax.ShapeDtypeStruct((_NC, _NP, d), jnp.float32),
        mesh=_mesh,
        scratch_types=[
            pltpu.VMEM((_NCH, _CSZ), jnp.int32),
            pltpu.VMEM((_NCH, _CSZ), jnp.int32),
            pltpu.VMEM((_CSZ, d), jnp.float32),
            pltpu.VMEM((_CSZ, d), jnp.float32),
            pltpu.VMEM_SHARED((_NP, d), jnp.float32),
            pltpu.SemaphoreType.DMA,
            pltpu.SemaphoreType.DMA,
        ],
        compiler_params=pltpu.CompilerParams(use_tc_tiling_on_sc=False),
    )
    def _sc_spmm(hp, src3, dst3, outp, src_all, dst_all, rows0, rows1, acc,
                 sem0, sem1):
        c = lax.axis_index("c")
        s = lax.axis_index("s")
        wid = s * _NC + c
        _zero_fill(rows0, _CSZ, d)
        _zero_acc_slice(rows0, acc, s * _RPT)
        pltpu.sync_copy(src3.at[wid], src_all)
        pltpu.sync_copy(dst3.at[wid], dst_all)
        plsc.subcore_barrier()
        pltpu.async_copy(hp.at[src_all.at[0]], rows0, sem0)
        def pair(p, _):
            g = 2 * p
            pltpu.async_copy(hp.at[src_all.at[g + 1]], rows1, sem1)
            pltpu.make_async_copy(hp.at[src_all.at[g]], rows0, sem0).wait()
            pltpu.sync_copy(rows0, acc.at[dst_all.at[g]], add=True)
            @pl.when(p < _NCH // 2 - 1)
            def _():
                pltpu.async_copy(hp.at[src_all.at[g + 2]], rows0, sem0)
            pltpu.make_async_copy(hp.at[src_all.at[g + 1]], rows1, sem1).wait()
            pltpu.sync_copy(rows1, acc.at[dst_all.at[g + 1]], add=True)
            return 0
        lax.fori_loop(0, _NCH // 2, pair, 0)
        plsc.subcore_barrier()
        pltpu.sync_copy(acc.at[pl.ds(s * _RPT, _RPT)],
                        outp.at[c, pl.ds(s * _RPT, _RPT)])
    return _sc_spmm


_sc_spmm_128 = _make_spmm(128)
_sc_spmm_64 = _make_spmm(64)


_R = 1000


def _tc1_body(x_ref, w_ref, d0_ref, d1_ref, hp_ref):
    deg = d0_ref[0, :, 0:1] + d1_ref[0, :, 0:1] + 1.0
    dis = lax.rsqrt(deg)
    h = jnp.dot(x_ref[...], w_ref[...], preferred_element_type=jnp.float32)
    hp_ref[...] = h * dis


def _tc_mid_body(s0_ref, s1_ref, hp_ref, d0_ref, d1_ref, b_ref, w_ref, out_ref):
    deg = d0_ref[0, :, 0:1] + d1_ref[0, :, 0:1] + 1.0
    dis = lax.rsqrt(deg)
    tot = s0_ref[0] + s1_ref[0] + hp_ref[...]
    h = jnp.maximum(tot * dis + b_ref[...], 0.0)
    out_ref[...] = jnp.dot(h, w_ref[...], preferred_element_type=jnp.float32) * dis


def _tc_out_body(s0_ref, s1_ref, hp_ref, d0_ref, d1_ref, b_ref, out_ref):
    deg = d0_ref[0, :, 0:1] + d1_ref[0, :, 0:1] + 1.0
    dis = lax.rsqrt(deg)
    out_ref[...] = (s0_ref[0] + s1_ref[0] + hp_ref[...]) * dis + b_ref[...]


def _row_spec(d):
    return pl.BlockSpec((_R, d), lambda i: (i, 0))


def _full_spec(shape):
    return pl.BlockSpec(shape, lambda i: tuple(0 for _ in shape))


def kernel(x, edge_index, W1, b1, W2, b2):
    src = edge_index[0].astype(jnp.int32)
    dst = edge_index[1].astype(jnp.int32)
    pad = _NW * _NCH * _CSZ - _E
    src3 = jnp.concatenate([src, jnp.zeros((pad,), jnp.int32)]).reshape(_NW, _NCH, _CSZ)
    dst3 = jnp.concatenate([dst, jnp.full((pad,), _N, jnp.int32)]).reshape(_NW, _NCH, _CSZ)

    degp = _sc_deg(dst3)

    def _part_spec(d, j):
        return pl.BlockSpec((1, _R, d), lambda i, j=j: (j, i, 0))

    hp1 = pl.pallas_call(
        _tc1_body,
        grid=(_N // _R,),
        in_specs=[_row_spec(128), _full_spec((128, 128)),
                  _part_spec(16, 0), _part_spec(16, 1)],
        out_specs=_row_spec(128),
        out_shape=jax.ShapeDtypeStruct((_N, 128), jnp.float32),
    )(x, W1, degp, degp)

    s1 = _sc_spmm_128(hp1, src3, dst3)

    hp2 = pl.pallas_call(
        _tc_mid_body,
        grid=(_N // _R,),
        in_specs=[
            _part_spec(128, 0),
            _part_spec(128, 1),
            _row_spec(128),
            _part_spec(16, 0),
            _part_spec(16, 1),
            _full_spec((1, 128)),
            _full_spec((128, 64)),
        ],
        out_specs=_row_spec(64),
        out_shape=jax.ShapeDtypeStruct((_N, 64), jnp.float32),
    )(s1, s1, hp1, degp, degp, b1.reshape(1, 128), W2)

    s2 = _sc_spmm_64(hp2, src3, dst3)

    out = pl.pallas_call(
        _tc_out_body,
        grid=(_N // _R,),
        in_specs=[
            _part_spec(64, 0),
            _part_spec(64, 1),
            _row_spec(64),
            _part_spec(16, 0),
            _part_spec(16, 1),
            _full_spec((1, 64)),
        ],
        out_specs=_row_spec(64),
        out_shape=jax.ShapeDtypeStruct((_N, 64), jnp.float32),
    )(s2, s2, hp2, degp, degp, b2.reshape(1, 64))
    return out
